# baseline (speedup 1.0000x reference)
import functools

import jax
import jax.numpy as jnp
from jax import lax
from jax.experimental import pallas as pl
from jax.experimental.pallas import tpu as pltpu
from jax.experimental.pallas import tpu_sc as plsc

B = 65536
LANES = 16
NSC = 2
NSUB = 16
CHUNK = 16384


def _minmax_body(x_ref, t_ref, o_ref, mn_ref, mx_ref):
    i = pl.program_id(0)
    xb = x_ref[...]
    tb = t_ref[...]
    mnv = jnp.minimum(jnp.min(xb, axis=0), jnp.min(tb, axis=0))[None, :]
    mxv = jnp.maximum(jnp.max(xb, axis=0), jnp.max(tb, axis=0))[None, :]

    @pl.when(i == 0)
    def _():
        mn_ref[...] = mnv
        mx_ref[...] = mxv

    @pl.when(i > 0)
    def _():
        mn_ref[...] = jnp.minimum(mn_ref[...], mnv)
        mx_ref[...] = jnp.maximum(mx_ref[...], mxv)

    @pl.when(i == pl.num_programs(0) - 1)
    def _():
        gmn = jnp.min(mn_ref[...])
        gmx = jnp.max(mx_ref[...])
        o_ref[...] = jnp.concatenate(
            [jnp.full((1, 128), gmn, jnp.float32),
             jnp.full((1, 128), gmx, jnp.float32)], axis=0)


def _minmax(xr, tr):
    rows = xr.shape[0]
    grid = 8
    blk = rows // grid
    return pl.pallas_call(
        _minmax_body,
        grid=(grid,),
        in_specs=[pl.BlockSpec((blk, 128), lambda i: (i, 0)),
                  pl.BlockSpec((blk, 128), lambda i: (i, 0))],
        out_specs=pl.BlockSpec((2, 128), lambda i: (0, 0)),
        out_shape=jax.ShapeDtypeStruct((2, 128), jnp.float32),
        scratch_shapes=[pltpu.VMEM((1, 128), jnp.float32),
                        pltpu.VMEM((1, 128), jnp.float32)],
    )(xr, tr)


def _make_hist_kernel(n):
    shard = n // NSUB
    nchunk = shard // CHUNK
    mesh = plsc.VectorSubcoreMesh(core_axis_name="c", subcore_axis_name="s")

    @functools.partial(
        pl.kernel,
        mesh=mesh,
        out_type=jax.ShapeDtypeStruct((NSC * NSUB * B,), jnp.int32),
        scratch_types=[
            pltpu.VMEM((CHUNK,), jnp.float32),
            pltpu.VMEM((CHUNK,), jnp.float32),
            pltpu.VMEM((B,), jnp.int32),
            pltpu.VMEM((LANES,), jnp.float32),
            pltpu.VMEM((LANES,), jnp.float32),
            pltpu.SemaphoreType.DMA,
            pltpu.SemaphoreType.DMA,
        ],
        compiler_params=pltpu.CompilerParams(needs_layout_passes=False),
    )
    def hist_kernel(x_hbm, t_hbm, gmn_hbm, gmx_hbm, out_hbm,
                    win0, win1, hist, gmn_v, gmx_v, sem_in0, sem_in1):
        c = lax.axis_index("c")
        s = lax.axis_index("s")

        pltpu.sync_copy(gmn_hbm, gmn_v)
        pltpu.sync_copy(gmx_hbm, gmx_v)
        gmin = gmn_v[...]
        rng = jnp.maximum(gmx_v[...] - gmin, jnp.float32(1e-30))
        invbw = jnp.float32(B) / rng

        def process(arr_ref):
            wins = (win0, win1)
            sin = (sem_in0, sem_in1)
            cp_in = [None, None]
            cp_in[0] = pltpu.async_copy(
                arr_ref.at[pl.ds(s * shard, CHUNK)], wins[0], sin[0])

            def zbody(i, _):
                for j in range(8):
                    hist[pl.ds((i * 8 + j) * LANES, LANES)] = jnp.zeros(
                        (LANES,), jnp.int32)
                return 0
            lax.fori_loop(0, B // (8 * LANES), zbody, 0)

            for k in range(nchunk):
                p = k & 1
                if k + 1 < nchunk:
                    base = s * shard + (k + 1) * CHUNK
                    cp_in[1 - p] = pltpu.async_copy(
                        arr_ref.at[pl.ds(base, CHUNK)], wins[1 - p],
                        sin[1 - p])
                cp_in[p].wait()
                win = wins[p]

                def bbody(r, _):
                    for j in range(8):
                        off = (r * 8 + j) * LANES
                        v = win[pl.ds(off, LANES)]
                        b = ((v - gmin) * invbw).astype(jnp.int32)
                        b = jnp.minimum(b, jnp.int32(B - 1))
                        plsc.addupdate_scatter(
                            hist, [b], jnp.ones((LANES,), jnp.int32))
                    return 0
                lax.fori_loop(0, CHUNK // (8 * LANES), bbody, 0)

        @pl.when(c == 0)
        def _():
            process(x_hbm)

        @pl.when(c == 1)
        def _():
            process(t_hbm)

        pltpu.sync_copy(hist, out_hbm.at[pl.ds((c * NSUB + s) * B, B)])

    return hist_kernel


def _w1_body(n, hx_ref, ht_ref, mm_ref, o_ref):
    d = (jnp.sum(hx_ref[...], axis=0)
         - jnp.sum(ht_ref[...], axis=0)).astype(jnp.float32)
    r128 = lax.broadcasted_iota(jnp.int32, (128, 128), 0)
    c128 = lax.broadcasted_iota(jnp.int32, (128, 128), 1)
    upper = (r128 <= c128).astype(jnp.float32)
    rowcum = lax.dot_general(
        d, upper, (((1,), (0,)), ((), ())),
        precision=lax.Precision.HIGHEST,
        preferred_element_type=jnp.float32)
    tot = rowcum[:, 127:128]
    r512 = lax.broadcasted_iota(jnp.int32, (512, 512), 0)
    c512 = lax.broadcasted_iota(jnp.int32, (512, 512), 1)
    strict_lower = (r512 > c512).astype(jnp.float32)
    off = lax.dot_general(
        strict_lower, tot, (((1,), (0,)), ((), ())),
        precision=lax.Precision.HIGHEST,
        preferred_element_type=jnp.float32)
    cum = rowcum + off
    ssum = jnp.sum(jnp.abs(cum))
    mm = mm_ref[...]
    rng = jnp.max(mm[1:2, :]) - jnp.min(mm[0:1, :])
    val = ssum * rng / jnp.float32(B) / jnp.float32(n)
    o_ref[...] = jnp.full((1, 128), val, jnp.float32)


def _w1(hx, ht, mm, n):
    return pl.pallas_call(
        functools.partial(_w1_body, n),
        out_shape=jax.ShapeDtypeStruct((1, 128), jnp.float32),
    )(hx, ht, mm)


def kernel(x, target):
    n = x.shape[0]
    xr = x.reshape(n // 128, 128)
    tr = target.reshape(n // 128, 128)
    mm = _minmax(xr, tr)
    gmn16 = lax.slice(mm, (0, 0), (1, LANES)).reshape(LANES)
    gmx16 = lax.slice(mm, (1, 0), (2, LANES)).reshape(LANES)
    hists = _make_hist_kernel(n)(x, target, gmn16, gmx16)
    h4 = hists.reshape(NSC, NSUB, B // 128, 128)
    out = _w1(h4[0], h4[1], mm, n)
    return out[0, 0]


# speedup vs baseline: 1.2192x; 1.2192x over previous
import functools

import jax
import jax.numpy as jnp
from jax import lax
from jax.experimental import pallas as pl
from jax.experimental.pallas import tpu as pltpu
from jax.experimental.pallas import tpu_sc as plsc

B = 65536
LANES = 16
NSC = 2
NSUB = 16
CHUNK = 8192


def _minmax_body(x_ref, t_ref, o_ref, mn_ref, mx_ref):
    i = pl.program_id(0)
    xb = x_ref[...]
    tb = t_ref[...]
    mnv = jnp.minimum(jnp.min(xb, axis=0), jnp.min(tb, axis=0))[None, :]
    mxv = jnp.maximum(jnp.max(xb, axis=0), jnp.max(tb, axis=0))[None, :]

    @pl.when(i == 0)
    def _():
        mn_ref[...] = mnv
        mx_ref[...] = mxv

    @pl.when(i > 0)
    def _():
        mn_ref[...] = jnp.minimum(mn_ref[...], mnv)
        mx_ref[...] = jnp.maximum(mx_ref[...], mxv)

    @pl.when(i == pl.num_programs(0) - 1)
    def _():
        gmn = jnp.min(mn_ref[...])
        gmx = jnp.max(mx_ref[...])
        o_ref[...] = jnp.concatenate(
            [jnp.full((1, 128), gmn, jnp.float32),
             jnp.full((1, 128), gmx, jnp.float32)], axis=0)


def _minmax(xr, tr):
    rows = xr.shape[0]
    grid = 8
    blk = rows // grid
    return pl.pallas_call(
        _minmax_body,
        grid=(grid,),
        in_specs=[pl.BlockSpec((blk, 128), lambda i: (i, 0)),
                  pl.BlockSpec((blk, 128), lambda i: (i, 0))],
        out_specs=pl.BlockSpec((2, 128), lambda i: (0, 0)),
        out_shape=jax.ShapeDtypeStruct((2, 128), jnp.float32),
        scratch_shapes=[pltpu.VMEM((1, 128), jnp.float32),
                        pltpu.VMEM((1, 128), jnp.float32)],
    )(xr, tr)


def _make_hist_kernel(n):
    shard = n // NSUB
    nchunk = shard // CHUNK
    slice_b = B // NSUB
    stream_set = frozenset(k for k in range(nchunk) if k % 8 not in (1, 3, 5))
    mesh = plsc.VectorSubcoreMesh(core_axis_name="c", subcore_axis_name="s")

    @functools.partial(
        pl.kernel,
        mesh=mesh,
        out_type=jax.ShapeDtypeStruct((NSC * (NSUB + 1) * B,), jnp.int32),
        scratch_types=[
            pltpu.VMEM((CHUNK,), jnp.float32),
            pltpu.VMEM((CHUNK,), jnp.float32),
            pltpu.VMEM((CHUNK,), jnp.int32),
            pltpu.VMEM((CHUNK,), jnp.int32),
            pltpu.VMEM((CHUNK,), jnp.int32),
            pltpu.VMEM((B // NSUB,), jnp.int32),
            pltpu.VMEM((B,), jnp.int32),
            pltpu.VMEM((LANES,), jnp.float32),
            pltpu.VMEM((LANES,), jnp.float32),
            pltpu.VMEM_SHARED((B,), jnp.int32),
            pltpu.SemaphoreType.DMA,
            pltpu.SemaphoreType.DMA,
            pltpu.SemaphoreType.DMA,
            pltpu.SemaphoreType.DMA,
        ],
        compiler_params=pltpu.CompilerParams(needs_layout_passes=False),
    )
    def hist_kernel(x_hbm, t_hbm, gmn_hbm, gmx_hbm, ones_hbm, zeros_hbm,
                    out_hbm,
                    win0, win1, idx0, idx1, ones_v, buf, histp, gmn_v, gmx_v,
                    hist_sh, sem_in0, sem_in1, sem_sc0, sem_sc1):
        c = lax.axis_index("c")
        s = lax.axis_index("s")

        pltpu.sync_copy(gmn_hbm, gmn_v)
        pltpu.sync_copy(gmx_hbm, gmx_v)
        pltpu.sync_copy(ones_hbm, ones_v)
        gmin = gmn_v[...]
        rng = jnp.maximum(gmx_v[...] - gmin, jnp.float32(1e-30))
        invbw = jnp.float32(B) / rng

        pltpu.sync_copy(zeros_hbm, buf)
        pltpu.sync_copy(buf, hist_sh.at[pl.ds(s * slice_b, slice_b)])
        plsc.subcore_barrier()

        def process(arr_ref):
            wins = (win0, win1)
            idxs = (idx0, idx1)
            sin = (sem_in0, sem_in1)
            ssc = (sem_sc0, sem_sc1)
            cp_in = [None, None]
            cp_sc = [None, None]
            cp_in[0] = pltpu.async_copy(
                arr_ref.at[pl.ds(s * shard, CHUNK)], wins[0], sin[0])

            def zbody(i, _):
                for j in range(8):
                    histp[pl.ds((i * 8 + j) * LANES, LANES)] = jnp.zeros(
                        (LANES,), jnp.int32)
                return 0
            lax.fori_loop(0, B // (8 * LANES), zbody, 0)

            for k in range(nchunk):
                p = k & 1
                if k + 1 < nchunk:
                    base = s * shard + (k + 1) * CHUNK
                    cp_in[1 - p] = pltpu.async_copy(
                        arr_ref.at[pl.ds(base, CHUNK)], wins[1 - p],
                        sin[1 - p])
                cp_in[p].wait()
                win = wins[p]
                if k in stream_set:
                    if cp_sc[p] is not None:
                        cp_sc[p].wait()
                    idx1d = idxs[p]

                    def sbody(r, _):
                        for j in range(8):
                            off = (r * 8 + j) * LANES
                            v = win[pl.ds(off, LANES)]
                            b = ((v - gmin) * invbw).astype(jnp.int32)
                            b = jnp.minimum(b, jnp.int32(B - 1))
                            idx1d[pl.ds(off, LANES)] = b
                        return 0
                    lax.fori_loop(0, CHUNK // (8 * LANES), sbody, 0)
                    cp_sc[p] = pltpu.async_copy(
                        ones_v, hist_sh.at[idx1d], ssc[p], add=True)
                else:
                    def pbody(r, _):
                        for j in range(8):
                            off = (r * 8 + j) * LANES
                            v = win[pl.ds(off, LANES)]
                            b = ((v - gmin) * invbw).astype(jnp.int32)
                            b = jnp.minimum(b, jnp.int32(B - 1))
                            plsc.addupdate_scatter(
                                histp, [b], jnp.ones((LANES,), jnp.int32))
                        return 0
                    lax.fori_loop(0, CHUNK // (8 * LANES), pbody, 0)
            for p in range(2):
                if cp_sc[p] is not None:
                    cp_sc[p].wait()

        @pl.when(c == 0)
        def _():
            process(x_hbm)

        @pl.when(c == 1)
        def _():
            process(t_hbm)

        pltpu.sync_copy(histp, out_hbm.at[pl.ds((c * (NSUB + 1) + s) * B, B)])
        plsc.subcore_barrier()
        pltpu.sync_copy(hist_sh.at[pl.ds(s * slice_b, slice_b)], buf)
        pltpu.sync_copy(
            buf,
            out_hbm.at[pl.ds((c * (NSUB + 1) + NSUB) * B + s * slice_b,
                             slice_b)])

    return hist_kernel


def _w1_body(n, hx_ref, ht_ref, mm_ref, o_ref):
    d = (jnp.sum(hx_ref[...], axis=0)
         - jnp.sum(ht_ref[...], axis=0)).astype(jnp.float32)
    r128 = lax.broadcasted_iota(jnp.int32, (128, 128), 0)
    c128 = lax.broadcasted_iota(jnp.int32, (128, 128), 1)
    upper = (r128 <= c128).astype(jnp.float32)
    rowcum = lax.dot_general(
        d, upper, (((1,), (0,)), ((), ())),
        precision=lax.Precision.HIGHEST,
        preferred_element_type=jnp.float32)
    tot = rowcum[:, 127:128]
    r512 = lax.broadcasted_iota(jnp.int32, (512, 512), 0)
    c512 = lax.broadcasted_iota(jnp.int32, (512, 512), 1)
    strict_lower = (r512 > c512).astype(jnp.float32)
    off = lax.dot_general(
        strict_lower, tot, (((1,), (0,)), ((), ())),
        precision=lax.Precision.HIGHEST,
        preferred_element_type=jnp.float32)
    cum = rowcum + off
    ssum = jnp.sum(jnp.abs(cum))
    mm = mm_ref[...]
    rng = jnp.max(mm[1:2, :]) - jnp.min(mm[0:1, :])
    val = ssum * rng / jnp.float32(B) / jnp.float32(n)
    o_ref[...] = jnp.full((1, 128), val, jnp.float32)


def _w1(hx, ht, mm, n):
    return pl.pallas_call(
        functools.partial(_w1_body, n),
        out_shape=jax.ShapeDtypeStruct((1, 128), jnp.float32),
    )(hx, ht, mm)


def kernel(x, target):
    n = x.shape[0]
    xr = x.reshape(n // 128, 128)
    tr = target.reshape(n // 128, 128)
    mm = _minmax(xr, tr)
    gmn16 = lax.slice(mm, (0, 0), (1, LANES)).reshape(LANES)
    gmx16 = lax.slice(mm, (1, 0), (2, LANES)).reshape(LANES)
    ones = jnp.ones((CHUNK,), jnp.int32)
    zeros = jnp.zeros((B // NSUB,), jnp.int32)
    hists = _make_hist_kernel(n)(x, target, gmn16, gmx16, ones, zeros)
    h4 = hists.reshape(NSC, NSUB + 1, B // 128, 128)
    out = _w1(h4[0], h4[1], mm, n)
    return out[0, 0]


# speedup vs baseline: 1.6037x; 1.3154x over previous
import functools

import jax
import jax.numpy as jnp
from jax import lax
from jax.experimental import pallas as pl
from jax.experimental.pallas import tpu as pltpu
from jax.experimental.pallas import tpu_sc as plsc

B = 65536
LANES = 16
NSC = 2
NSUB = 16
CHUNK = 16384


def _minmax_body(x_ref, t_ref, o_ref, mn_ref, mx_ref):
    i = pl.program_id(0)
    xb = x_ref[...]
    tb = t_ref[...]
    mnv = jnp.minimum(jnp.min(xb, axis=0), jnp.min(tb, axis=0))[None, :]
    mxv = jnp.maximum(jnp.max(xb, axis=0), jnp.max(tb, axis=0))[None, :]

    @pl.when(i == 0)
    def _():
        mn_ref[...] = mnv
        mx_ref[...] = mxv

    @pl.when(i > 0)
    def _():
        mn_ref[...] = jnp.minimum(mn_ref[...], mnv)
        mx_ref[...] = jnp.maximum(mx_ref[...], mxv)

    @pl.when(i == pl.num_programs(0) - 1)
    def _():
        gmn = jnp.min(mn_ref[...])
        gmx = jnp.max(mx_ref[...])
        o_ref[...] = jnp.concatenate(
            [jnp.full((1, 128), gmn, jnp.float32),
             jnp.full((1, 128), gmx, jnp.float32)], axis=0)


def _minmax(xr, tr):
    rows = xr.shape[0]
    grid = 8
    blk = rows // grid
    return pl.pallas_call(
        _minmax_body,
        grid=(grid,),
        in_specs=[pl.BlockSpec((blk, 128), lambda i: (i, 0)),
                  pl.BlockSpec((blk, 128), lambda i: (i, 0))],
        out_specs=pl.BlockSpec((2, 128), lambda i: (0, 0)),
        out_shape=jax.ShapeDtypeStruct((2, 128), jnp.float32),
        scratch_shapes=[pltpu.VMEM((1, 128), jnp.float32),
                        pltpu.VMEM((1, 128), jnp.float32)],
    )(xr, tr)


def _make_hist_kernel(n):
    shard = n // NSUB
    nchunk = shard // CHUNK
    slice_b = B // NSUB
    mesh = plsc.VectorSubcoreMesh(core_axis_name="c", subcore_axis_name="s")

    @functools.partial(
        pl.kernel,
        mesh=mesh,
        out_type=jax.ShapeDtypeStruct((NSC * B,), jnp.int32),
        scratch_types=[
            pltpu.VMEM((CHUNK,), jnp.float32),
            pltpu.VMEM((CHUNK,), jnp.float32),
            pltpu.VMEM((CHUNK,), jnp.int32),
            pltpu.VMEM((CHUNK,), jnp.int32),
            pltpu.VMEM((CHUNK,), jnp.int32),
            pltpu.VMEM((slice_b,), jnp.int32),
            pltpu.VMEM((2 * LANES,), jnp.float32),
            pltpu.VMEM_SHARED((B,), jnp.int32),
            pltpu.SemaphoreType.DMA,
            pltpu.SemaphoreType.DMA,
            pltpu.SemaphoreType.DMA,
            pltpu.SemaphoreType.DMA,
            pltpu.SemaphoreType.DMA,
        ],
    )
    def hist_kernel(x_hbm, t_hbm, mm_hbm, ones_hbm, zeros_hbm,
                    out_hbm,
                    win0, win1, idx0, idx1, ones_v, buf, mm_v, hist,
                    sem_in0, sem_in1, sem_sc0, sem_sc1, sem_on):
        c = lax.axis_index("c")
        s = lax.axis_index("s")

        cp_ones = pltpu.async_copy(ones_hbm, ones_v, sem_on)
        pltpu.sync_copy(mm_hbm, mm_v)
        gmin = mm_v[pl.ds(0, LANES)]
        rng = jnp.maximum(mm_v[pl.ds(LANES, LANES)] - gmin,
                          jnp.float32(1e-30))
        invbw = jnp.float32(B) / rng

        pltpu.sync_copy(zeros_hbm, buf)
        pltpu.sync_copy(buf, hist.at[pl.ds(s * slice_b, slice_b)])
        plsc.subcore_barrier()

        def process(arr_ref):
            wins = (win0, win1)
            idxs = (idx0, idx1)
            sin = (sem_in0, sem_in1)
            ssc = (sem_sc0, sem_sc1)
            cp_in = [None, None]
            cp_sc = [None, None]
            cp_in[0] = pltpu.async_copy(
                arr_ref.at[pl.ds(s * shard, CHUNK)], wins[0], sin[0])
            for k in range(nchunk):
                p = k & 1
                if k + 1 < nchunk:
                    base = s * shard + (k + 1) * CHUNK
                    cp_in[1 - p] = pltpu.async_copy(
                        arr_ref.at[pl.ds(base, CHUNK)], wins[1 - p],
                        sin[1 - p])
                cp_in[p].wait()
                if cp_sc[p] is not None:
                    cp_sc[p].wait()
                win = wins[p]
                idx1d = idxs[p]

                def bbody(r, _):
                    for j in range(8):
                        off = (r * 8 + j) * LANES
                        v = win[pl.ds(off, LANES)]
                        b = ((v - gmin) * invbw).astype(jnp.int32)
                        b = jnp.minimum(b, jnp.int32(B - 1))
                        idx1d[pl.ds(off, LANES)] = b
                    return 0
                lax.fori_loop(0, CHUNK // (8 * LANES), bbody, 0)
                if k == 0:
                    cp_ones.wait()
                cp_sc[p] = pltpu.async_copy(
                    ones_v, hist.at[idx1d], ssc[p], add=True)
            for p in range(2):
                if cp_sc[p] is not None:
                    cp_sc[p].wait()

        @pl.when(c == 0)
        def _():
            process(x_hbm)

        @pl.when(c == 1)
        def _():
            process(t_hbm)

        plsc.subcore_barrier()
        pltpu.sync_copy(hist.at[pl.ds(s * slice_b, slice_b)], buf)
        pltpu.sync_copy(buf, out_hbm.at[pl.ds(c * B + s * slice_b, slice_b)])

    return hist_kernel


def _w1_body(n, hx_ref, ht_ref, mm_ref, o_ref):
    d = (hx_ref[...].astype(jnp.float32)
         - ht_ref[...].astype(jnp.float32))
    r128 = lax.broadcasted_iota(jnp.int32, (128, 128), 0)
    c128 = lax.broadcasted_iota(jnp.int32, (128, 128), 1)
    upper = (r128 <= c128).astype(jnp.float32)
    rowcum = lax.dot_general(
        d, upper, (((1,), (0,)), ((), ())),
        precision=lax.Precision.HIGHEST,
        preferred_element_type=jnp.float32)
    tot = rowcum[:, 127:128]
    r512 = lax.broadcasted_iota(jnp.int32, (512, 512), 0)
    c512 = lax.broadcasted_iota(jnp.int32, (512, 512), 1)
    strict_lower = (r512 > c512).astype(jnp.float32)
    off = lax.dot_general(
        strict_lower, tot, (((1,), (0,)), ((), ())),
        precision=lax.Precision.HIGHEST,
        preferred_element_type=jnp.float32)
    cum = rowcum + off
    ssum = jnp.sum(jnp.abs(cum))
    mm = mm_ref[...]
    rng = jnp.max(mm[1:2, :]) - jnp.min(mm[0:1, :])
    val = ssum * rng / jnp.float32(B) / jnp.float32(n)
    o_ref[...] = jnp.full((1, 128), val, jnp.float32)


def _w1(hx, ht, mm, n):
    return pl.pallas_call(
        functools.partial(_w1_body, n),
        out_shape=jax.ShapeDtypeStruct((1, 128), jnp.float32),
    )(hx, ht, mm)


def kernel(x, target):
    n = x.shape[0]
    xr = x.reshape(n // 128, 128)
    tr = target.reshape(n // 128, 128)
    mm = _minmax(xr, tr)
    mm32 = lax.slice(mm, (0, 0), (2, LANES)).reshape(2 * LANES)
    ones = jnp.ones((CHUNK,), jnp.int32)
    zeros = jnp.zeros((B // NSUB,), jnp.int32)
    hists = _make_hist_kernel(n)(x, target, mm32, ones, zeros)
    h3 = hists.reshape(NSC, B // 128, 128)
    out = _w1(h3[0], h3[1], mm, n)
    return out[0, 0]
